# Initial kernel scaffold; baseline (speedup 1.0000x reference)
import functools

import jax
import jax.numpy as jnp
from jax import lax
from jax.experimental import pallas as pl
from jax.experimental.pallas import tpu as pltpu
from jax.experimental.pallas import tpu_sc as plsc

NC = 2
NS = 16
NW = NC * NS
K = 128
DEG_W = 16


def _mesh():
    return plsc.VectorSubcoreMesh(core_axis_name="c", subcore_axis_name="s")


def _zero_rows(buf, d):
    @pl.loop(0, K)
    def _(i):
        @pl.loop(0, d, step=16)
        def _(j):
            buf[i, pl.ds(j, 16)] = jnp.zeros((16,), jnp.float32)


@functools.lru_cache(maxsize=None)
def _make_sc_degree(e_pad, n_pad):
    epw = e_pad // NW
    nchunk = epw // K
    rpt = n_pad // NS

    @functools.partial(
        pl.kernel,
        out_type=jax.ShapeDtypeStruct((NC, n_pad, DEG_W), jnp.float32),
        mesh=_mesh(),
        scratch_types=[
            pltpu.VMEM((K,), jnp.int32),
            pltpu.VMEM((K, DEG_W), jnp.float32),
            pltpu.VMEM_SHARED((n_pad, DEG_W), jnp.float32),
        ],
    )
    def sc_degree(dst_hbm, out_hbm, dst_v, buf_v, acc):
        c = lax.axis_index("c")
        s = lax.axis_index("s")
        wid = s * NC + c
        base_row = s * rpt
        _zero_rows(buf_v, DEG_W)

        @pl.loop(0, rpt, step=K)
        def _(r):
            pltpu.sync_copy(buf_v, acc.at[pl.ds(base_row + r, K)])

        plsc.subcore_barrier()

        @pl.loop(0, K)
        def _(i):
            buf_v[i, pl.ds(0, 16)] = jnp.ones((16,), jnp.float32)

        @pl.loop(0, nchunk)
        def _(j):
            e0 = wid * epw + j * K
            pltpu.sync_copy(dst_hbm.at[pl.ds(e0, K)], dst_v)
            pltpu.sync_copy(buf_v, acc.at[dst_v], add=True)

        plsc.subcore_barrier()
        pltpu.sync_copy(acc.at[pl.ds(base_row, rpt)],
                        out_hbm.at[c].at[pl.ds(base_row, rpt)])

    return sc_degree


@functools.lru_cache(maxsize=None)
def _make_sc_scatter(d, e_pad, n_pad):
    epw = e_pad // NW
    nchunk = epw // K
    rpt = n_pad // NS

    @functools.partial(
        pl.kernel,
        out_type=jax.ShapeDtypeStruct((NC, n_pad, d), jnp.float32),
        mesh=_mesh(),
        scratch_types=[
            pltpu.VMEM((K,), jnp.int32),
            pltpu.VMEM((K,), jnp.int32),
            pltpu.VMEM((K, d), jnp.float32),
            pltpu.VMEM_SHARED((n_pad, d), jnp.float32),
            pltpu.SemaphoreType.DMA,
        ],
    )
    def sc_scatter(h_hbm, src_hbm, dst_hbm, out_hbm,
                   src_v, dst_v, rows_v, acc, sem):
        c = lax.axis_index("c")
        s = lax.axis_index("s")
        wid = s * NC + c
        base_row = s * rpt
        _zero_rows(rows_v, d)

        @pl.loop(0, rpt, step=K)
        def _(r):
            pltpu.sync_copy(rows_v, acc.at[pl.ds(base_row + r, K)])

        plsc.subcore_barrier()

        @pl.loop(0, nchunk)
        def _(j):
            e0 = wid * epw + j * K
            pltpu.sync_copy(src_hbm.at[pl.ds(e0, K)], src_v)
            pltpu.sync_copy(dst_hbm.at[pl.ds(e0, K)], dst_v)
            pltpu.async_copy(h_hbm.at[src_v], rows_v, sem).wait()
            pltpu.sync_copy(rows_v, acc.at[dst_v], add=True)

        plsc.subcore_barrier()
        pltpu.sync_copy(acc.at[pl.ds(base_row, rpt)],
                        out_hbm.at[c].at[pl.ds(base_row, rpt)])

    return sc_scatter


def _dinv_col(dp_ref, rows):
    d0 = dp_ref[0, :rows, 0:1]
    d1 = dp_ref[1, :rows, 0:1]
    return lax.rsqrt(jnp.maximum(d0 + d1, 1.0))


def _tc_layer1(x, w1, deg_p):
    n, d_hid = x.shape[0], w1.shape[1]

    def body(x_ref, w_ref, dp_ref, o_ref):
        dinv = _dinv_col(dp_ref, n)
        h = jnp.dot(x_ref[...], w_ref[...],
                    preferred_element_type=jnp.float32)
        o_ref[...] = h * dinv

    return pl.pallas_call(
        body, out_shape=jax.ShapeDtypeStruct((n, d_hid), jnp.float32),
    )(x, w1, deg_p)


def _tc_layer2(agg1_p, deg_p, b1, w2):
    n_pad = agg1_p.shape[1]
    d_out = w2.shape[1]

    def body(ap_ref, dp_ref, b_ref, w_ref, o_ref):
        dinv = _dinv_col(dp_ref, n_pad)
        a = (ap_ref[0] + ap_ref[1]) * dinv + b_ref[...]
        h1 = jnp.maximum(a, 0.0)
        h2 = jnp.dot(h1, w_ref[...], preferred_element_type=jnp.float32)
        o_ref[...] = h2 * dinv

    return pl.pallas_call(
        body, out_shape=jax.ShapeDtypeStruct((n_pad, d_out), jnp.float32),
    )(agg1_p, deg_p, b1, w2)


def _tc_final(agg2_p, deg_p, b2):
    n_pad, d_out = agg2_p.shape[1], agg2_p.shape[2]

    def body(ap_ref, dp_ref, b_ref, o_ref):
        dinv = _dinv_col(dp_ref, n_pad)
        o_ref[...] = (ap_ref[0] + ap_ref[1]) * dinv + b_ref[...]

    return pl.pallas_call(
        body, out_shape=jax.ShapeDtypeStruct((n_pad, d_out), jnp.float32),
    )(agg2_p, deg_p, b2)


def kernel(x, edge_index, W1, b1, W2, b2):
    n = x.shape[0]
    e = edge_index.shape[1]
    chunk_total = NW * K
    e_pad = ((e + chunk_total - 1) // chunk_total) * chunk_total
    n_pad = ((n + (NS * K) - 1) // (NS * K)) * (NS * K)

    src = edge_index[0]
    dst = edge_index[1]
    pad = e_pad - e
    if pad:
        src = jnp.concatenate([src, jnp.zeros((pad,), jnp.int32)])
        dst = jnp.concatenate([dst, jnp.full((pad,), n, jnp.int32)])

    deg_p = _make_sc_degree(e_pad, n_pad)(dst)
    hs = _tc_layer1(x, W1, deg_p)
    agg1_p = _make_sc_scatter(W1.shape[1], e_pad, n_pad)(hs, src, dst)
    hs2 = _tc_layer2(agg1_p, deg_p, b1, W2)
    agg2_p = _make_sc_scatter(W2.shape[1], e_pad, n_pad)(hs2, src, dst)
    out_pad = _tc_final(agg2_p, deg_p, b2)
    return out_pad[:n]


# speedup vs baseline: 10.7290x; 10.7290x over previous
import functools

import jax
import jax.numpy as jnp
from jax import lax
from jax.experimental import pallas as pl
from jax.experimental.pallas import tpu as pltpu
from jax.experimental.pallas import tpu_sc as plsc

NC = 2
NS = 16
NW = NC * NS
K = 128
DEG_W = 16


def _mesh():
    return plsc.VectorSubcoreMesh(core_axis_name="c", subcore_axis_name="s")


_SC_PARAMS = pltpu.CompilerParams(use_tc_tiling_on_sc=False)


def _zero_rows(buf, d):
    @pl.loop(0, K)
    def _(i):
        @pl.loop(0, d, step=16)
        def _(j):
            buf[i, pl.ds(j, 16)] = jnp.zeros((16,), jnp.float32)


@functools.lru_cache(maxsize=None)
def _make_sc_degree(e_pad, n_pad):
    epw = e_pad // NW
    nchunk = epw // K
    rpt = n_pad // NS

    @functools.partial(
        pl.kernel,
        out_type=jax.ShapeDtypeStruct((NC, n_pad, DEG_W), jnp.float32),
        mesh=_mesh(),
        scratch_types=[
            pltpu.VMEM((K,), jnp.int32),
            pltpu.VMEM((K, DEG_W), jnp.float32),
            pltpu.VMEM_SHARED((n_pad, DEG_W), jnp.float32),
        ],
        compiler_params=_SC_PARAMS,
    )
    def sc_degree(dst_hbm, out_hbm, dst_v, buf_v, acc):
        c = lax.axis_index("c")
        s = lax.axis_index("s")
        wid = s * NC + c
        base_row = s * rpt
        _zero_rows(buf_v, DEG_W)

        @pl.loop(0, rpt, step=K)
        def _(r):
            pltpu.sync_copy(buf_v, acc.at[pl.ds(base_row + r, K)])

        plsc.subcore_barrier()

        @pl.loop(0, K)
        def _(i):
            buf_v[i, pl.ds(0, 16)] = jnp.ones((16,), jnp.float32)

        @pl.loop(0, nchunk)
        def _(j):
            e0 = wid * epw + j * K
            pltpu.sync_copy(dst_hbm.at[pl.ds(e0, K)], dst_v)
            pltpu.sync_copy(buf_v, acc.at[dst_v], add=True)

        plsc.subcore_barrier()
        pltpu.sync_copy(acc.at[pl.ds(base_row, rpt)],
                        out_hbm.at[c].at[pl.ds(base_row, rpt)])

    return sc_degree


@functools.lru_cache(maxsize=None)
def _make_sc_scatter(d, e_pad, n_pad):
    epw = e_pad // NW
    nchunk = epw // K
    rpt = n_pad // NS

    @functools.partial(
        pl.kernel,
        out_type=jax.ShapeDtypeStruct((NC, n_pad, d), jnp.float32),
        mesh=_mesh(),
        scratch_types=[
            pltpu.VMEM((K,), jnp.int32),
            pltpu.VMEM((K,), jnp.int32),
            pltpu.VMEM((K, d), jnp.float32),
            pltpu.VMEM_SHARED((n_pad, d), jnp.float32),
            pltpu.SemaphoreType.DMA,
        ],
        compiler_params=_SC_PARAMS,
    )
    def sc_scatter(h_hbm, src_hbm, dst_hbm, out_hbm,
                   src_v, dst_v, rows_v, acc, sem):
        c = lax.axis_index("c")
        s = lax.axis_index("s")
        wid = s * NC + c
        base_row = s * rpt
        _zero_rows(rows_v, d)

        @pl.loop(0, rpt, step=K)
        def _(r):
            pltpu.sync_copy(rows_v, acc.at[pl.ds(base_row + r, K)])

        plsc.subcore_barrier()

        @pl.loop(0, nchunk)
        def _(j):
            e0 = wid * epw + j * K
            pltpu.sync_copy(src_hbm.at[pl.ds(e0, K)], src_v)
            pltpu.sync_copy(dst_hbm.at[pl.ds(e0, K)], dst_v)
            pltpu.async_copy(h_hbm.at[src_v], rows_v, sem).wait()
            pltpu.sync_copy(rows_v, acc.at[dst_v], add=True)

        plsc.subcore_barrier()
        pltpu.sync_copy(acc.at[pl.ds(base_row, rpt)],
                        out_hbm.at[c].at[pl.ds(base_row, rpt)])

    return sc_scatter


def _dinv_col(dp_ref, rows):
    d0 = dp_ref[0, :rows, 0:1]
    d1 = dp_ref[1, :rows, 0:1]
    return lax.rsqrt(jnp.maximum(d0 + d1, 1.0))


def _tc_layer1(x, w1, deg_p):
    n, d_hid = x.shape[0], w1.shape[1]

    def body(x_ref, w_ref, dp_ref, o_ref):
        dinv = _dinv_col(dp_ref, n)
        h = jnp.dot(x_ref[...], w_ref[...],
                    preferred_element_type=jnp.float32)
        o_ref[...] = h * dinv

    return pl.pallas_call(
        body, out_shape=jax.ShapeDtypeStruct((n, d_hid), jnp.float32),
    )(x, w1, deg_p)


def _tc_layer2(agg1_p, deg_p, b1, w2):
    n_pad = agg1_p.shape[1]
    d_out = w2.shape[1]

    def body(ap_ref, dp_ref, b_ref, w_ref, o_ref):
        dinv = _dinv_col(dp_ref, n_pad)
        a = (ap_ref[0] + ap_ref[1]) * dinv + b_ref[...]
        h1 = jnp.maximum(a, 0.0)
        h2 = jnp.dot(h1, w_ref[...], preferred_element_type=jnp.float32)
        o_ref[...] = h2 * dinv

    return pl.pallas_call(
        body, out_shape=jax.ShapeDtypeStruct((n_pad, d_out), jnp.float32),
    )(agg1_p, deg_p, b1, w2)


def _tc_final(agg2_p, deg_p, b2):
    n_pad, d_out = agg2_p.shape[1], agg2_p.shape[2]

    def body(ap_ref, dp_ref, b_ref, o_ref):
        dinv = _dinv_col(dp_ref, n_pad)
        o_ref[...] = (ap_ref[0] + ap_ref[1]) * dinv + b_ref[...]

    return pl.pallas_call(
        body, out_shape=jax.ShapeDtypeStruct((n_pad, d_out), jnp.float32),
    )(agg2_p, deg_p, b2)


def kernel(x, edge_index, W1, b1, W2, b2):
    n = x.shape[0]
    e = edge_index.shape[1]
    chunk_total = NW * K
    e_pad = ((e + chunk_total - 1) // chunk_total) * chunk_total
    n_pad = ((n + (NS * K) - 1) // (NS * K)) * (NS * K)

    src = edge_index[0]
    dst = edge_index[1]
    pad = e_pad - e
    if pad:
        src = jnp.concatenate([src, jnp.zeros((pad,), jnp.int32)])
        dst = jnp.concatenate([dst, jnp.full((pad,), n, jnp.int32)])

    deg_p = _make_sc_degree(e_pad, n_pad)(dst)
    hs = _tc_layer1(x, W1, deg_p)
    agg1_p = _make_sc_scatter(W1.shape[1], e_pad, n_pad)(hs, src, dst)
    hs2 = _tc_layer2(agg1_p, deg_p, b1, W2)
    agg2_p = _make_sc_scatter(W2.shape[1], e_pad, n_pad)(hs2, src, dst)
    out_pad = _tc_final(agg2_p, deg_p, b2)
    return out_pad[:n]
